# baseline (speedup 1.0000x reference)
import functools

import jax
import jax.numpy as jnp
from jax import lax
from jax.experimental import pallas as pl
from jax.experimental.pallas import tpu as pltpu
from jax.experimental.pallas import tpu_sc as plsc

_HID = 1024
_B = 4
_S = 2048
_EPS = 1e-12

_NW = 32
_HALF = (_B * _S) // 2
_TPW = _HALF // _NW
_C = 16
_NCHUNK = _TPW // _C


def _sc_gather_body(ids_hbm, word_hbm, out_hbm,
                    ids_v, rows0, rows1, rows2, rows3,
                    gs0, gs1, gs2, gs3, os0, os1, os2, os3):
    wid = lax.axis_index("s") * 2 + lax.axis_index("c")
    t0 = wid * _TPW

    rows = (rows0, rows1, rows2, rows3)
    gsems = (gs0, gs1, gs2, gs3)
    osems = (os0, os1, os2, os3)

    pltpu.sync_copy(ids_hbm.at[pl.ds(t0, _TPW)], ids_v)

    def start_gather(cc, b):
        pltpu.async_copy(word_hbm.at[ids_v.at[pl.ds(cc * _C, _C)]],
                         rows[b], gsems[b])

    start_gather(0, 0)
    start_gather(1, 1)

    def chunk_quad(cc0, carry):
        for j in range(4):
            cc = cc0 + j
            pltpu.make_async_copy(word_hbm.at[pl.ds(0, _C)],
                                  rows[j], gsems[j]).wait()
            pltpu.async_copy(rows[j], out_hbm.at[pl.ds(t0 + cc * _C, _C)],
                             osems[j])

            jn = (j + 2) % 4
            @pl.when((cc + 2 >= 4) & (cc + 2 < _NCHUNK))
            def _():
                pltpu.make_async_copy(rows[jn], out_hbm.at[pl.ds(0, _C)],
                                      osems[jn]).wait()

            @pl.when(cc + 2 < _NCHUNK)
            def _():
                start_gather(cc + 2, jn)
        return carry

    lax.fori_loop(0, _NCHUNK // 4, lambda i, c: chunk_quad(4 * i, c), 0)

    for j in range(4):
        pltpu.make_async_copy(rows[j], out_hbm.at[pl.ds(0, _C)],
                              osems[j]).wait()


def _ln_body(g_ref, pos_ref, tok_ref, gamma_ref, beta_ref, o_ref):
    e = g_ref[...] + pos_ref[...] + tok_ref[0:1, :]
    m = jnp.mean(e, axis=1, keepdims=True)
    d = e - m
    v = jnp.mean(d * d, axis=1, keepdims=True)
    o_ref[...] = (d * lax.rsqrt(v + _EPS)) * gamma_ref[0:1, :] \
        + beta_ref[0:1, :]


@jax.jit
def _embeddings_ln(ids_flat, W_word, W_pos, W_tok, gamma2d, beta2d):
    mesh = plsc.VectorSubcoreMesh(core_axis_name="c", subcore_axis_name="s")
    sc_gather = functools.partial(
        pl.kernel,
        mesh=mesh,
        out_type=jax.ShapeDtypeStruct((_HALF, _HID), jnp.float32),
        scratch_types=[
            pltpu.VMEM((_TPW,), jnp.int32),
            pltpu.VMEM((_C, _HID), jnp.float32),
            pltpu.VMEM((_C, _HID), jnp.float32),
            pltpu.VMEM((_C, _HID), jnp.float32),
            pltpu.VMEM((_C, _HID), jnp.float32),
            pltpu.SemaphoreType.DMA,
            pltpu.SemaphoreType.DMA,
            pltpu.SemaphoreType.DMA,
            pltpu.SemaphoreType.DMA,
            pltpu.SemaphoreType.DMA,
            pltpu.SemaphoreType.DMA,
            pltpu.SemaphoreType.DMA,
            pltpu.SemaphoreType.DMA,
        ],
    )(_sc_gather_body)

    tc_ln = functools.partial(
        pl.pallas_call,
        _ln_body,
        grid=(2,),
        in_specs=[
            pl.BlockSpec((_S, _HID), lambda b: (b, 0)),
            pl.BlockSpec((_S, _HID), lambda b: (0, 0)),
            pl.BlockSpec((2, _HID), lambda b: (0, 0)),
            pl.BlockSpec((1, _HID), lambda b: (0, 0)),
            pl.BlockSpec((1, _HID), lambda b: (0, 0)),
        ],
        out_specs=pl.BlockSpec((_S, _HID), lambda b: (b, 0)),
        out_shape=jax.ShapeDtypeStruct((_HALF, _HID), jnp.float32),
    )

    g0 = sc_gather(ids_flat[:_HALF], W_word)
    g1 = sc_gather(ids_flat[_HALF:], W_word)
    o0 = tc_ln()(g0, W_pos, W_tok, gamma2d, beta2d)
    o1 = tc_ln()(g1, W_pos, W_tok, gamma2d, beta2d)
    return jnp.concatenate([o0, o1], axis=0)


def kernel(input_ids, W_word, W_pos, W_tok, gamma, beta):
    ids_flat = input_ids.reshape(-1)
    out = _embeddings_ln(ids_flat, W_word, W_pos, W_tok,
                         gamma.reshape(1, _HID), beta.reshape(1, _HID))
    return out.reshape(_B, _S, _HID)


# speedup vs baseline: 1.3254x; 1.3254x over previous
import functools

import jax
import jax.numpy as jnp
from jax import lax
from jax.experimental import pallas as pl
from jax.experimental.pallas import tpu as pltpu
from jax.experimental.pallas import tpu_sc as plsc

_HID = 1024
_B = 4
_S = 2048
_EPS = 1e-12

_NW = 32
_TPW = (_B * _S) // _NW
_C = 16
_NCHUNK = _TPW // _C
_PBLK = 2048
_NPB = _S // _PBLK


def _sc_gather_body(ids_hbm, word_hbm, out_hbm,
                    ids_v, rows0, rows1, rows2, rows3,
                    gs0, gs1, gs2, gs3, os0, os1, os2, os3):
    wid = lax.axis_index("s") * 2 + lax.axis_index("c")
    t0 = wid * _TPW

    rows = (rows0, rows1, rows2, rows3)
    gsems = (gs0, gs1, gs2, gs3)
    osems = (os0, os1, os2, os3)

    pltpu.sync_copy(ids_hbm.at[pl.ds(t0, _TPW)], ids_v)

    def start_gather(cc, b):
        pltpu.async_copy(word_hbm.at[ids_v.at[pl.ds(cc * _C, _C)]],
                         rows[b], gsems[b])

    start_gather(0, 0)
    start_gather(1, 1)

    def chunk_quad(cc0, carry):
        for j in range(4):
            cc = cc0 + j
            pltpu.make_async_copy(word_hbm.at[pl.ds(0, _C)],
                                  rows[j], gsems[j]).wait()
            pltpu.async_copy(rows[j], out_hbm.at[pl.ds(t0 + cc * _C, _C)],
                             osems[j])

            jn = (j + 2) % 4
            @pl.when((cc + 2 >= 4) & (cc + 2 < _NCHUNK))
            def _():
                pltpu.make_async_copy(rows[jn], out_hbm.at[pl.ds(0, _C)],
                                      osems[jn]).wait()

            @pl.when(cc + 2 < _NCHUNK)
            def _():
                start_gather(cc + 2, jn)
        return carry

    lax.fori_loop(0, _NCHUNK // 4, lambda i, c: chunk_quad(4 * i, c), 0)

    for j in range(4):
        pltpu.make_async_copy(rows[j], out_hbm.at[pl.ds(0, _C)],
                              osems[j]).wait()


def _ln_body(g_ref, pos_ref, tok_ref, gamma_ref, beta_ref, o_ref):
    e = g_ref[...] + pos_ref[...] + tok_ref[0:1, :]
    m = jnp.mean(e, axis=1, keepdims=True)
    d = e - m
    v = jnp.mean(d * d, axis=1, keepdims=True)
    o_ref[...] = (d * lax.rsqrt(v + _EPS)) * gamma_ref[0:1, :] \
        + beta_ref[0:1, :]


@jax.jit
def _embeddings_ln(ids_flat, W_word, W_pos, W_tok, gamma2d, beta2d):
    mesh = plsc.VectorSubcoreMesh(core_axis_name="c", subcore_axis_name="s")
    gathered = functools.partial(
        pl.kernel,
        mesh=mesh,
        out_type=jax.ShapeDtypeStruct((_B * _S, _HID), jnp.float32),
        scratch_types=[
            pltpu.VMEM((_TPW,), jnp.int32),
            pltpu.VMEM((_C, _HID), jnp.float32),
            pltpu.VMEM((_C, _HID), jnp.float32),
            pltpu.VMEM((_C, _HID), jnp.float32),
            pltpu.VMEM((_C, _HID), jnp.float32),
            pltpu.SemaphoreType.DMA,
            pltpu.SemaphoreType.DMA,
            pltpu.SemaphoreType.DMA,
            pltpu.SemaphoreType.DMA,
            pltpu.SemaphoreType.DMA,
            pltpu.SemaphoreType.DMA,
            pltpu.SemaphoreType.DMA,
            pltpu.SemaphoreType.DMA,
        ],
    )(_sc_gather_body)(ids_flat, W_word)

    return pl.pallas_call(
        _ln_body,
        grid=(_NPB, _B),
        in_specs=[
            pl.BlockSpec((_PBLK, _HID), lambda p, b: (b * _NPB + p, 0)),
            pl.BlockSpec((_PBLK, _HID), lambda p, b: (p, 0)),
            pl.BlockSpec((2, _HID), lambda p, b: (0, 0)),
            pl.BlockSpec((1, _HID), lambda p, b: (0, 0)),
            pl.BlockSpec((1, _HID), lambda p, b: (0, 0)),
        ],
        out_specs=pl.BlockSpec((_PBLK, _HID), lambda p, b: (b * _NPB + p, 0)),
        out_shape=jax.ShapeDtypeStruct((_B * _S, _HID), jnp.float32),
    )(gathered, W_pos, W_tok, gamma2d, beta2d)


def kernel(input_ids, W_word, W_pos, W_tok, gamma, beta):
    ids_flat = input_ids.reshape(-1)
    out = _embeddings_ln(ids_flat, W_word, W_pos, W_tok,
                         gamma.reshape(1, _HID), beta.reshape(1, _HID))
    return out.reshape(_B, _S, _HID)
